# baseline (speedup 1.0000x reference)
import jax
import jax.numpy as jnp
from jax import lax
from jax.experimental import pallas as pl
from jax.experimental.pallas import tpu as pltpu
from jax.experimental.pallas import tpu_sc as plsc

_B = 16384
_D = 64
_NCAT = 4
_NSUB = 12
_NCOMBO = _NCAT * _NSUB
_NC = 2
_NS = 16
_NW = _NC * _NS
_RPW = _B // _NW
_L = 16
_CH = 128
_NCH = _RPW // _CH
_CG = _CH // _L


def _body(cat_hbm, sub_hbm, nzt_hbm, catt_hbm, subt_hbm, outt_hbm,
          ci_v, si_v, catt_v, subt_v, combo_v, rad_lut_v, emb_v,
          nz0_v, nz1_v, ot0_v, ot1_v, sem, si0, si1, so0, so1):
    wid = lax.axis_index("s") * _NC + lax.axis_index("c")
    base = wid * _RPW
    nz = [nz0_v, nz1_v]
    ot = [ot0_v, ot1_v]
    sin = [si0, si1]
    son = [so0, so1]

    cps = [
        pltpu.async_copy(cat_hbm.at[pl.ds(base, _RPW)], ci_v, sem),
        pltpu.async_copy(sub_hbm.at[pl.ds(base, _RPW)], si_v, sem),
        pltpu.async_copy(catt_hbm, catt_v, sem),
        pltpu.async_copy(subt_hbm, subt_v, sem),
    ]
    in_cp = pltpu.async_copy(nzt_hbm.at[:, pl.ds(base, _CH)], nz0_v, si0)
    for cp in cps:
        cp.wait()

    lane = lax.iota(jnp.int32, _L)
    lane48 = lane * _NCOMBO

    crow8 = [[8.0 * catt_v[c, pl.ds(j * _L, _L)] for j in range(4)]
             for c in range(_NCAT)]
    srow10 = [[10.0 * subt_v[s, pl.ds(j * _L, _L)] for j in range(4)]
              for s in range(_NSUB)]
    for cid in range(_NCOMBO):
        c, s = cid // _NSUB, cid % _NSUB
        for j in range(4):
            plsc.store_scatter(combo_v,
                               [lane48 + (j * _L * _NCOMBO + cid)],
                               crow8[c][j] + srow10[s][j])

    rad_lut_v[...] = 0.3 + (0.4 / 3.0) * (lane % 3).astype(jnp.float32)

    def make_group(nz_ref, ot_ref, ioff):
        def group(g, carry):
            r0 = g * _L
            civ = ci_v[pl.ds(ioff + r0, _L)]
            siv = si_v[pl.ds(ioff + r0, _L)]
            cidv = civ * _NSUB + siv
            radv = plsc.load_gather(rad_lut_v, [siv])
            sqs = [jnp.zeros((_L,), jnp.float32) for _ in range(4)]
            for db in range(0, _D, 16):
                ebl = [(plsc.load_gather(combo_v,
                                         [cidv + (db + u) * _NCOMBO])
                        + nz_ref[db + u, pl.ds(r0, _L)]) for u in range(16)]
                for u in range(16):
                    sqs[u % 4] = sqs[u % 4] + ebl[u] * ebl[u]
                for u in range(16):
                    emb_v[pl.ds((db + u) * _L, _L)] = ebl[u]
            sq = (sqs[0] + sqs[1]) + (sqs[2] + sqs[3])
            s = jnp.maximum(sq, 1e-4)
            i = plsc.bitcast(s, jnp.int32)
            i = jnp.int32(0x5F3759DF) - lax.shift_right_arithmetic(i, 1)
            t = plsc.bitcast(i, jnp.float32)
            for _ in range(3):
                t = t * (1.5 - 0.5 * s * t * t)
            x = 2e-5 * t
            fvec = radv * (t * (1.0 - x))
            for db in range(0, _D, 16):
                ebl = [emb_v[pl.ds((db + u) * _L, _L)] for u in range(16)]
                for u in range(16):
                    ot_ref[db + u, pl.ds(r0, _L)] = ebl[u] * fvec
            return carry
        return group

    out_cp = [None, None]
    for c in range(_NCH):
        buf = c % 2
        if c + 1 < _NCH:
            nxt = pltpu.async_copy(
                nzt_hbm.at[:, pl.ds(base + (c + 1) * _CH, _CH)],
                nz[1 - buf], sin[1 - buf])
        in_cp.wait()
        if out_cp[buf] is not None:
            out_cp[buf].wait()
        lax.fori_loop(0, _CG, make_group(nz[buf], ot[buf], c * _CH), 0)
        out_cp[buf] = pltpu.async_copy(
            ot[buf], outt_hbm.at[:, pl.ds(base + c * _CH, _CH)], son[buf])
        if c + 1 < _NCH:
            in_cp = nxt
    out_cp[0].wait()
    out_cp[1].wait()


def kernel(cat_idx, subcat_idx, noise, category_dirs, subcategory_dirs):
    mesh = plsc.VectorSubcoreMesh(core_axis_name="c", subcore_axis_name="s")
    run = pl.kernel(
        _body,
        mesh=mesh,
        out_type=jax.ShapeDtypeStruct((_D, _B), jnp.float32),
        compiler_params=pltpu.CompilerParams(needs_layout_passes=False),
        scratch_types=[
            pltpu.VMEM((_RPW,), jnp.int32),
            pltpu.VMEM((_RPW,), jnp.int32),
            pltpu.VMEM((_NCAT, _D), jnp.float32),
            pltpu.VMEM((_NSUB, _D), jnp.float32),
            pltpu.VMEM((_D * _NCOMBO,), jnp.float32),
            pltpu.VMEM((_L,), jnp.float32),
            pltpu.VMEM((_D * _L,), jnp.float32),
            pltpu.VMEM((_D, _CH), jnp.float32),
            pltpu.VMEM((_D, _CH), jnp.float32),
            pltpu.VMEM((_D, _CH), jnp.float32),
            pltpu.VMEM((_D, _CH), jnp.float32),
            pltpu.SemaphoreType.DMA,
            pltpu.SemaphoreType.DMA,
            pltpu.SemaphoreType.DMA,
            pltpu.SemaphoreType.DMA,
            pltpu.SemaphoreType.DMA,
        ],
    )
    out_t = run(cat_idx, subcat_idx, noise.T, category_dirs,
                subcategory_dirs)
    return out_t.T
